# Initial kernel scaffold; baseline (speedup 1.0000x reference)
import jax
import jax.numpy as jnp
from jax.experimental import pallas as pl


def kernel(x, edge_index, batch, Wl1, b1, Wr1, Wl2, b2, Wr2, Wl3, b3, Wr3):
    raise NotImplementedError("write your pallas kernel here")


# speedup vs baseline: 6.5333x; 6.5333x over previous
import jax
import jax.numpy as jnp
from jax import lax
from jax.experimental import pallas as pl
from jax.experimental.pallas import tpu as pltpu
from jax.experimental.pallas import tpu_sc as plsc

N = 10000
E = 320000
DIN = 128
H1 = 256
H2 = 192
C = 40

NC = 2
NS = 16
CH = 128
CHUNKS = -(-E // (NS * CH))
EPT = CHUNKS * CH
EPAD = NS * EPT
NP = 10112
RPT = NP // NS

DH1 = 80
DH2 = 96
DH3 = 32


def _make_segsum(DH):
  mesh = plsc.VectorSubcoreMesh(core_axis_name="c", subcore_axis_name="s")

  def body(ulo_hbm, uhi_hbm, src_hbm, dst_hbm, z_hbm, out_hbm,
           src_v, dst_v, rows_v, agg_sh, sem):
    c = lax.axis_index("c")
    s = lax.axis_index("s")
    pltpu.sync_copy(z_hbm, agg_sh.at[pl.ds(s * RPT, RPT)])
    pltpu.sync_copy(src_hbm.at[s], src_v)
    pltpu.sync_copy(dst_hbm.at[s], dst_v)
    plsc.subcore_barrier()

    def step(j, carry):
      @pl.when(c == 0)
      def _():
        pltpu.async_copy(ulo_hbm.at[src_v.at[j]], rows_v, sem).wait()

      @pl.when(c == 1)
      def _():
        pltpu.async_copy(uhi_hbm.at[src_v.at[j]], rows_v, sem).wait()

      pltpu.sync_copy(rows_v, agg_sh.at[dst_v.at[j]], add=True)
      return carry

    lax.fori_loop(0, CHUNKS, step, 0)
    plsc.subcore_barrier()
    pltpu.sync_copy(agg_sh.at[pl.ds(s * RPT, RPT)],
                    out_hbm.at[c, pl.ds(s * RPT, RPT)])

  return pl.kernel(
      body,
      out_type=jax.ShapeDtypeStruct((NC, NP, DH), jnp.float32),
      mesh=mesh,
      compiler_params=pltpu.CompilerParams(use_tc_tiling_on_sc=False),
      scratch_types=[
          pltpu.VMEM((CHUNKS, CH), jnp.int32),
          pltpu.VMEM((CHUNKS, CH), jnp.int32),
          pltpu.VMEM((CH, DH), jnp.float32),
          pltpu.VMEM_SHARED((NP, DH), jnp.float32),
          pltpu.SemaphoreType.DMA,
      ],
  )


_segsum_d1 = _make_segsum(DH1)
_segsum_d2 = _make_segsum(DH2)
_segsum_d3 = _make_segsum(DH3)

BN = 1000


def _l1_body(x_ref, p0_ref, p1_ref, wl1_ref, wr1_ref, wl2a_ref, wl2b_ref,
             b1_ref, h1_ref, u2a_ref, u2b_ref, inv_ref):
  s = jnp.concatenate([p0_ref[:, :], p1_ref[:, :]], axis=1)
  cnt = s[:, DIN:DIN + 1]
  inv = 1.0 / jnp.maximum(cnt, 1.0)
  mean = s[:, :DIN] * inv
  h1 = jnp.tanh(
      jnp.dot(mean, wl1_ref[:, :], preferred_element_type=jnp.float32)
      + b1_ref[:, :]
      + jnp.dot(x_ref[:, :], wr1_ref[:, :], preferred_element_type=jnp.float32))
  h1_ref[:, :] = h1
  u2a_ref[:, :] = jnp.dot(h1, wl2a_ref[:, :], preferred_element_type=jnp.float32)
  u2b_ref[:, :] = jnp.dot(h1, wl2b_ref[:, :], preferred_element_type=jnp.float32)
  inv_ref[:, :] = jnp.broadcast_to(inv, (BN, 8))


def _l2_body(p0_ref, p1_ref, h1_ref, inv_ref, wr2_ref, b2_ref, wl3a_ref,
             wl3b_ref, h2_ref, u3a_ref, u3b_ref):
  agg = jnp.concatenate([p0_ref[:, :], p1_ref[:, :]], axis=1) * inv_ref[:, 0:1]
  h2 = jax.nn.relu(
      agg + b2_ref[:, :]
      + jnp.dot(h1_ref[:, :], wr2_ref[:, :], preferred_element_type=jnp.float32))
  h2_ref[:, :] = h2
  u3a_ref[:, :] = jnp.dot(h2, wl3a_ref[:, :], preferred_element_type=jnp.float32)
  u3b_ref[:, :] = jnp.dot(h2, wl3b_ref[:, :], preferred_element_type=jnp.float32)


def _l3_body(p0_ref, p1_ref, h2_ref, inv_ref, wr3_ref, b3_ref, out_ref):
  agg = jnp.concatenate([p0_ref[:, :], p1_ref[:, :C - DH3]], axis=1)
  out_ref[:, :] = jax.nn.sigmoid(
      agg * inv_ref[:, 0:1] + b3_ref[:, :]
      + jnp.dot(h2_ref[:, :], wr3_ref[:, :], preferred_element_type=jnp.float32))


def _row_spec(d):
  return pl.BlockSpec((BN, d), lambda i: (i, 0))


def _full_spec(shape):
  return pl.BlockSpec(shape, lambda i: (0,) * len(shape))


_layer1 = pl.pallas_call(
    _l1_body,
    grid=(N // BN,),
    in_specs=[
        _row_spec(DIN), _row_spec(DH1), _row_spec(DH1),
        _full_spec((DIN, H1)), _full_spec((DIN, H1)),
        _full_spec((H1, DH2)), _full_spec((H1, DH2)),
        _full_spec((1, H1)),
    ],
    out_specs=[_row_spec(H1), _row_spec(DH2), _row_spec(DH2), _row_spec(8)],
    out_shape=[
        jax.ShapeDtypeStruct((N, H1), jnp.float32),
        jax.ShapeDtypeStruct((N, DH2), jnp.float32),
        jax.ShapeDtypeStruct((N, DH2), jnp.float32),
        jax.ShapeDtypeStruct((N, 8), jnp.float32),
    ],
)

_layer2 = pl.pallas_call(
    _l2_body,
    grid=(N // BN,),
    in_specs=[
        _row_spec(DH2), _row_spec(DH2), _row_spec(H1), _row_spec(8),
        _full_spec((H1, H2)), _full_spec((1, H2)),
        _full_spec((H2, DH3)), _full_spec((H2, DH3)),
    ],
    out_specs=[_row_spec(H2), _row_spec(DH3), _row_spec(DH3)],
    out_shape=[
        jax.ShapeDtypeStruct((N, H2), jnp.float32),
        jax.ShapeDtypeStruct((N, DH3), jnp.float32),
        jax.ShapeDtypeStruct((N, DH3), jnp.float32),
    ],
)

_layer3 = pl.pallas_call(
    _l3_body,
    grid=(N // BN,),
    in_specs=[
        _row_spec(DH3), _row_spec(DH3), _row_spec(H2), _row_spec(8),
        _full_spec((H2, C)), _full_spec((1, C)),
    ],
    out_specs=_row_spec(C),
    out_shape=jax.ShapeDtypeStruct((N, C), jnp.float32),
)


def kernel(x, edge_index, batch, Wl1, b1, Wr1, Wl2, b2, Wr2, Wl3, b3, Wr3):
  f32 = jnp.float32
  src = edge_index[0]
  dst = edge_index[1]
  pad = EPAD - E
  srcp = jnp.concatenate([src, jnp.zeros((pad,), jnp.int32)]).reshape(
      NS, CHUNKS, CH)
  dstp = jnp.concatenate([dst, jnp.full((pad,), N, jnp.int32)]).reshape(
      NS, CHUNKS, CH)

  xa_lo = x[:, :DH1]
  xa_hi = jnp.concatenate(
      [x[:, DH1:DIN], jnp.ones((N, 1), f32),
       jnp.zeros((N, 2 * DH1 - DIN - 1), f32)], axis=1)
  p1 = _segsum_d1(xa_lo, xa_hi, srcp, dstp, jnp.zeros((RPT, DH1), f32))
  h1, u2a, u2b, inv = _layer1(x, p1[0, :N], p1[1, :N], Wl1, Wr1,
                              Wl2[:, :DH2], Wl2[:, DH2:], b1.reshape(1, H1))

  p2 = _segsum_d2(u2a, u2b, srcp, dstp, jnp.zeros((RPT, DH2), f32))
  wl3p = jnp.pad(Wl3, ((0, 0), (0, 2 * DH3 - C)))
  h2, u3a, u3b = _layer2(p2[0, :N], p2[1, :N], h1, inv, Wr2,
                         b2.reshape(1, H2), wl3p[:, :DH3], wl3p[:, DH3:])

  p3 = _segsum_d3(u3a, u3b, srcp, dstp, jnp.zeros((RPT, DH3), f32))
  return _layer3(p3[0, :N], p3[1, :N], h2, inv, Wr3, b3.reshape(1, C))


# speedup vs baseline: 7.7018x; 1.1789x over previous
import jax
import jax.numpy as jnp
from jax import lax
from jax.experimental import pallas as pl
from jax.experimental.pallas import tpu as pltpu
from jax.experimental.pallas import tpu_sc as plsc

N = 10000
E = 320000
DIN = 128
H1 = 256
H2 = 192
C = 40

NC = 2
NS = 16
CH = 128
NBUF = 4
HALF = NBUF // 2
CHUNKS = 158
TAIL = CHUNKS % NBUF
EPT = CHUNKS * CH
EPAD = NS * EPT
NP = 10112
RPT = NP // NS
ZR = 64

DH1 = 80
DH2 = 96
DH3 = 32


def _make_segsum(DH):
  mesh = plsc.VectorSubcoreMesh(core_axis_name="c", subcore_axis_name="s")

  def body(ulo_hbm, uhi_hbm, packed_hbm, z_hbm, out_hbm,
           packed_v, si0, si1, si2, si3, di0, di1, di2, di3,
           r0, r1, r2, r3, agg_sh,
           g0, g1, g2, g3, s0, s1, s2, s3):
    rows = (r0, r1, r2, r3)
    src_i = (si0, si1, si2, si3)
    dst_i = (di0, di1, di2, di3)
    sg = (g0, g1, g2, g3)
    ss = (s0, s1, s2, s3)
    c = lax.axis_index("c")
    tid = lax.axis_index("s")
    for i in range(RPT // ZR):
      pltpu.sync_copy(z_hbm, agg_sh.at[pl.ds(tid * RPT + i * ZR, ZR)])
    rem = RPT % ZR
    if rem:
      pltpu.sync_copy(z_hbm.at[pl.ds(0, rem)],
                      agg_sh.at[pl.ds(tid * RPT + (RPT // ZR) * ZR, rem)])
    pltpu.sync_copy(packed_hbm.at[tid], packed_v)
    plsc.subcore_barrier()

    def load_idx(jj, b):
      row = packed_v.at[jj]
      for k in range(CH // 16):
        v = row[pl.ds(k * 16, 16)]
        src_i[b][pl.ds(k * 16, 16)] = v & 0x3FFF
        dst_i[b][pl.ds(k * 16, 16)] = lax.shift_right_logical(v, 14)

    def issue_gather(jj, b):
      load_idx(jj, b)

      @pl.when(c == 0)
      def _():
        pltpu.async_copy(ulo_hbm.at[src_i[b]], rows[b], sg[b])

      @pl.when(c == 1)
      def _():
        pltpu.async_copy(uhi_hbm.at[src_i[b]], rows[b], sg[b])

    for b in range(HALF):
      issue_gather(b, b)

    def outer(t, carry):
      for b in range(NBUF):
        jj = t * NBUF + b
        b2 = (b + HALF) % NBUF

        @pl.when(jj >= HALF)
        def _(b2=b2):
          pltpu.make_async_copy(
              rows[b2], agg_sh.at[dst_i[b2]], ss[b2]).wait()

        issue_gather(jj + HALF, b2)

        pltpu.make_async_copy(ulo_hbm.at[src_i[b]], rows[b], sg[b]).wait()
        pltpu.async_copy(rows[b], agg_sh.at[dst_i[b]], ss[b], add=True)
      return carry

    lax.fori_loop(0, CHUNKS // NBUF, outer, 0)
    for jj in range(CHUNKS - TAIL, CHUNKS):
      b = jj % NBUF
      pltpu.make_async_copy(ulo_hbm.at[src_i[b]], rows[b], sg[b]).wait()
      pltpu.async_copy(rows[b], agg_sh.at[dst_i[b]], ss[b], add=True)
    for b in range(NBUF):
      pltpu.make_async_copy(rows[b], agg_sh.at[dst_i[b]], ss[b]).wait()
    plsc.subcore_barrier()
    pltpu.sync_copy(agg_sh.at[pl.ds(tid * RPT, RPT)],
                    out_hbm.at[c, pl.ds(tid * RPT, RPT)])

  return pl.kernel(
      body,
      out_type=jax.ShapeDtypeStruct((NC, NP, DH), jnp.float32),
      mesh=mesh,
      compiler_params=pltpu.CompilerParams(use_tc_tiling_on_sc=False),
      scratch_types=(
          [pltpu.VMEM((CHUNKS, CH), jnp.int32)]
          + [pltpu.VMEM((CH,), jnp.int32) for _ in range(2 * NBUF)]
          + [pltpu.VMEM((CH, DH), jnp.float32) for _ in range(NBUF)]
          + [pltpu.VMEM_SHARED((NP, DH), jnp.float32)]
          + [pltpu.SemaphoreType.DMA for _ in range(2 * NBUF)]
      ),
  )


_segsum_d1 = _make_segsum(DH1)
_segsum_d2 = _make_segsum(DH2)
_segsum_d3 = _make_segsum(DH3)

BN = 1000


def _l1_body(x_ref, p0_ref, p1_ref, wl1_ref, wr1_ref, wl2a_ref, wl2b_ref,
             b1_ref, h1_ref, u2a_ref, u2b_ref, inv_ref):
  s = jnp.concatenate([p0_ref[:, :], p1_ref[:, :]], axis=1)
  cnt = s[:, DIN:DIN + 1]
  inv = 1.0 / jnp.maximum(cnt, 1.0)
  mean = s[:, :DIN] * inv
  h1 = jnp.tanh(
      jnp.dot(mean, wl1_ref[:, :], preferred_element_type=jnp.float32)
      + b1_ref[:, :]
      + jnp.dot(x_ref[:, :], wr1_ref[:, :], preferred_element_type=jnp.float32))
  h1_ref[:, :] = h1
  u2a_ref[:, :] = jnp.dot(h1, wl2a_ref[:, :], preferred_element_type=jnp.float32)
  u2b_ref[:, :] = jnp.dot(h1, wl2b_ref[:, :], preferred_element_type=jnp.float32)
  inv_ref[:, :] = jnp.broadcast_to(inv, (BN, 8))


def _l2_body(p0_ref, p1_ref, h1_ref, inv_ref, wr2_ref, b2_ref, wl3a_ref,
             wl3b_ref, h2_ref, u3a_ref, u3b_ref):
  agg = jnp.concatenate([p0_ref[:, :], p1_ref[:, :]], axis=1) * inv_ref[:, 0:1]
  h2 = jax.nn.relu(
      agg + b2_ref[:, :]
      + jnp.dot(h1_ref[:, :], wr2_ref[:, :], preferred_element_type=jnp.float32))
  h2_ref[:, :] = h2
  u3a_ref[:, :] = jnp.dot(h2, wl3a_ref[:, :], preferred_element_type=jnp.float32)
  u3b_ref[:, :] = jnp.dot(h2, wl3b_ref[:, :], preferred_element_type=jnp.float32)


def _l3_body(p0_ref, p1_ref, h2_ref, inv_ref, wr3_ref, b3_ref, out_ref):
  agg = jnp.concatenate([p0_ref[:, :], p1_ref[:, :C - DH3]], axis=1)
  out_ref[:, :] = jax.nn.sigmoid(
      agg * inv_ref[:, 0:1] + b3_ref[:, :]
      + jnp.dot(h2_ref[:, :], wr3_ref[:, :], preferred_element_type=jnp.float32))


def _row_spec(d):
  return pl.BlockSpec((BN, d), lambda i: (i, 0))


def _full_spec(shape):
  return pl.BlockSpec(shape, lambda i: (0,) * len(shape))


_layer1 = pl.pallas_call(
    _l1_body,
    grid=(N // BN,),
    in_specs=[
        _row_spec(DIN), _row_spec(DH1), _row_spec(DH1),
        _full_spec((DIN, H1)), _full_spec((DIN, H1)),
        _full_spec((H1, DH2)), _full_spec((H1, DH2)),
        _full_spec((1, H1)),
    ],
    out_specs=[_row_spec(H1), _row_spec(DH2), _row_spec(DH2), _row_spec(8)],
    out_shape=[
        jax.ShapeDtypeStruct((N, H1), jnp.float32),
        jax.ShapeDtypeStruct((N, DH2), jnp.float32),
        jax.ShapeDtypeStruct((N, DH2), jnp.float32),
        jax.ShapeDtypeStruct((N, 8), jnp.float32),
    ],
)

_layer2 = pl.pallas_call(
    _l2_body,
    grid=(N // BN,),
    in_specs=[
        _row_spec(DH2), _row_spec(DH2), _row_spec(H1), _row_spec(8),
        _full_spec((H1, H2)), _full_spec((1, H2)),
        _full_spec((H2, DH3)), _full_spec((H2, DH3)),
    ],
    out_specs=[_row_spec(H2), _row_spec(DH3), _row_spec(DH3)],
    out_shape=[
        jax.ShapeDtypeStruct((N, H2), jnp.float32),
        jax.ShapeDtypeStruct((N, DH3), jnp.float32),
        jax.ShapeDtypeStruct((N, DH3), jnp.float32),
    ],
)

_layer3 = pl.pallas_call(
    _l3_body,
    grid=(N // BN,),
    in_specs=[
        _row_spec(DH3), _row_spec(DH3), _row_spec(H2), _row_spec(8),
        _full_spec((H2, C)), _full_spec((1, C)),
    ],
    out_specs=_row_spec(C),
    out_shape=jax.ShapeDtypeStruct((N, C), jnp.float32),
)


def kernel(x, edge_index, batch, Wl1, b1, Wr1, Wl2, b2, Wr2, Wl3, b3, Wr3):
  f32 = jnp.float32
  src = edge_index[0]
  dst = edge_index[1]
  pad = EPAD - E
  packed = jnp.left_shift(dst, 14) | src
  packedp = jnp.concatenate(
      [packed, jnp.full((pad,), N << 14, jnp.int32)]).reshape(NS, CHUNKS, CH)

  xa_lo = x[:, :DH1]
  xa_hi = jnp.concatenate(
      [x[:, DH1:DIN], jnp.ones((N, 1), f32),
       jnp.zeros((N, 2 * DH1 - DIN - 1), f32)], axis=1)
  p1 = _segsum_d1(xa_lo, xa_hi, packedp, jnp.zeros((ZR, DH1), f32))
  h1, u2a, u2b, inv = _layer1(x, p1[0, :N], p1[1, :N], Wl1, Wr1,
                              Wl2[:, :DH2], Wl2[:, DH2:], b1.reshape(1, H1))

  p2 = _segsum_d2(u2a, u2b, packedp, jnp.zeros((ZR, DH2), f32))
  wl3p = jnp.pad(Wl3, ((0, 0), (0, 2 * DH3 - C)))
  h2, u3a, u3b = _layer2(p2[0, :N], p2[1, :N], h1, inv, Wr2,
                         b2.reshape(1, H2), wl3p[:, :DH3], wl3p[:, DH3:])

  p3 = _segsum_d3(u3a, u3b, packedp, jnp.zeros((ZR, DH3), f32))
  return _layer3(p3[0, :N], p3[1, :N], h2, inv, Wr3, b3.reshape(1, C))


# speedup vs baseline: 7.7105x; 1.0011x over previous
import jax
import jax.numpy as jnp
from jax import lax
from jax.experimental import pallas as pl
from jax.experimental.pallas import tpu as pltpu
from jax.experimental.pallas import tpu_sc as plsc

N = 10000
E = 320000
DIN = 128
H1 = 256
H2 = 192
C = 40

NC = 2
NS = 16
CH = 128
CHUNKS = 158
EPT = CHUNKS * CH
EPAD = NS * EPT
NP = 10112
RPT = NP // NS
ZR = 64

DH1 = 80
DH2 = 96
DH3 = 32


def _make_segsum(DH, NBUF):
  HALF = NBUF // 2
  TAIL = CHUNKS % NBUF
  mesh = plsc.VectorSubcoreMesh(core_axis_name="c", subcore_axis_name="s")

  def body(ulo_hbm, uhi_hbm, packed_hbm, z_hbm, out_hbm, *scratch):
    packed_v = scratch[0]
    src_i = scratch[1:1 + NBUF]
    dst_i = scratch[1 + NBUF:1 + 2 * NBUF]
    rows = scratch[1 + 2 * NBUF:1 + 3 * NBUF]
    agg_sh = scratch[1 + 3 * NBUF]
    sg = scratch[2 + 3 * NBUF:2 + 4 * NBUF]
    ss = scratch[2 + 4 * NBUF:2 + 5 * NBUF]
    c = lax.axis_index("c")
    tid = lax.axis_index("s")
    for i in range(RPT // ZR):
      pltpu.sync_copy(z_hbm, agg_sh.at[pl.ds(tid * RPT + i * ZR, ZR)])
    rem = RPT % ZR
    if rem:
      pltpu.sync_copy(z_hbm.at[pl.ds(0, rem)],
                      agg_sh.at[pl.ds(tid * RPT + (RPT // ZR) * ZR, rem)])
    pltpu.sync_copy(packed_hbm.at[tid], packed_v)
    plsc.subcore_barrier()

    def load_idx(jj, b):
      row = packed_v.at[jj]
      for k in range(CH // 16):
        v = row[pl.ds(k * 16, 16)]
        src_i[b][pl.ds(k * 16, 16)] = v & 0x3FFF
        dst_i[b][pl.ds(k * 16, 16)] = lax.shift_right_logical(v, 14)

    def issue_gather(jj, b):
      load_idx(jj, b)

      @pl.when(c == 0)
      def _():
        pltpu.async_copy(ulo_hbm.at[src_i[b]], rows[b], sg[b])

      @pl.when(c == 1)
      def _():
        pltpu.async_copy(uhi_hbm.at[src_i[b]], rows[b], sg[b])

    for b in range(HALF):
      issue_gather(b, b)

    def outer(t, carry):
      for b in range(NBUF):
        jj = t * NBUF + b
        b2 = (b + HALF) % NBUF

        @pl.when(jj >= NBUF - HALF)
        def _(b2=b2):
          pltpu.make_async_copy(
              rows[b2], agg_sh.at[dst_i[b2]], ss[b2]).wait()

        issue_gather(jj + HALF, b2)

        pltpu.make_async_copy(ulo_hbm.at[src_i[b]], rows[b], sg[b]).wait()
        pltpu.async_copy(rows[b], agg_sh.at[dst_i[b]], ss[b], add=True)
      return carry

    lax.fori_loop(0, CHUNKS // NBUF, outer, 0)
    for jj in range(CHUNKS - TAIL, CHUNKS):
      b = jj % NBUF
      b2 = (b + HALF) % NBUF
      if jj + HALF < CHUNKS:
        pltpu.make_async_copy(rows[b2], agg_sh.at[dst_i[b2]], ss[b2]).wait()
        issue_gather(jj + HALF, b2)
      pltpu.make_async_copy(ulo_hbm.at[src_i[b]], rows[b], sg[b]).wait()
      pltpu.async_copy(rows[b], agg_sh.at[dst_i[b]], ss[b], add=True)
    for b in range(NBUF):
      pltpu.make_async_copy(rows[b], agg_sh.at[dst_i[b]], ss[b]).wait()
    plsc.subcore_barrier()
    pltpu.sync_copy(agg_sh.at[pl.ds(tid * RPT, RPT)],
                    out_hbm.at[c, pl.ds(tid * RPT, RPT)])

  return pl.kernel(
      body,
      out_type=jax.ShapeDtypeStruct((NC, NP, DH), jnp.float32),
      mesh=mesh,
      compiler_params=pltpu.CompilerParams(use_tc_tiling_on_sc=False),
      scratch_types=(
          [pltpu.VMEM((CHUNKS, CH), jnp.int32)]
          + [pltpu.VMEM((CH,), jnp.int32) for _ in range(2 * NBUF)]
          + [pltpu.VMEM((CH, DH), jnp.float32) for _ in range(NBUF)]
          + [pltpu.VMEM_SHARED((NP, DH), jnp.float32)]
          + [pltpu.SemaphoreType.DMA for _ in range(2 * NBUF)]
      ),
  )


_segsum_d1 = _make_segsum(DH1, 4)
_segsum_d2 = _make_segsum(DH2, 3)
_segsum_d3 = _make_segsum(DH3, 8)

BN = 1000


def _l1_body(x_ref, p0_ref, p1_ref, wl1_ref, wr1_ref, wl2a_ref, wl2b_ref,
             b1_ref, h1_ref, u2a_ref, u2b_ref, inv_ref):
  s = jnp.concatenate([p0_ref[:, :], p1_ref[:, :]], axis=1)
  cnt = s[:, DIN:DIN + 1]
  inv = 1.0 / jnp.maximum(cnt, 1.0)
  mean = s[:, :DIN] * inv
  h1 = jnp.tanh(
      jnp.dot(mean, wl1_ref[:, :], preferred_element_type=jnp.float32)
      + b1_ref[:, :]
      + jnp.dot(x_ref[:, :], wr1_ref[:, :], preferred_element_type=jnp.float32))
  h1_ref[:, :] = h1
  u2a_ref[:, :] = jnp.dot(h1, wl2a_ref[:, :], preferred_element_type=jnp.float32)
  u2b_ref[:, :] = jnp.dot(h1, wl2b_ref[:, :], preferred_element_type=jnp.float32)
  inv_ref[:, :] = jnp.broadcast_to(inv, (BN, 8))


def _l2_body(p0_ref, p1_ref, h1_ref, inv_ref, wr2_ref, b2_ref, wl3a_ref,
             wl3b_ref, h2_ref, u3a_ref, u3b_ref):
  agg = jnp.concatenate([p0_ref[:, :], p1_ref[:, :]], axis=1) * inv_ref[:, 0:1]
  h2 = jax.nn.relu(
      agg + b2_ref[:, :]
      + jnp.dot(h1_ref[:, :], wr2_ref[:, :], preferred_element_type=jnp.float32))
  h2_ref[:, :] = h2
  u3a_ref[:, :] = jnp.dot(h2, wl3a_ref[:, :], preferred_element_type=jnp.float32)
  u3b_ref[:, :] = jnp.dot(h2, wl3b_ref[:, :], preferred_element_type=jnp.float32)


def _l3_body(p0_ref, p1_ref, h2_ref, inv_ref, wr3_ref, b3_ref, out_ref):
  agg = jnp.concatenate([p0_ref[:, :], p1_ref[:, :C - DH3]], axis=1)
  out_ref[:, :] = jax.nn.sigmoid(
      agg * inv_ref[:, 0:1] + b3_ref[:, :]
      + jnp.dot(h2_ref[:, :], wr3_ref[:, :], preferred_element_type=jnp.float32))


def _row_spec(d):
  return pl.BlockSpec((BN, d), lambda i: (i, 0))


def _full_spec(shape):
  return pl.BlockSpec(shape, lambda i: (0,) * len(shape))


_layer1 = pl.pallas_call(
    _l1_body,
    grid=(N // BN,),
    in_specs=[
        _row_spec(DIN), _row_spec(DH1), _row_spec(DH1),
        _full_spec((DIN, H1)), _full_spec((DIN, H1)),
        _full_spec((H1, DH2)), _full_spec((H1, DH2)),
        _full_spec((1, H1)),
    ],
    out_specs=[_row_spec(H1), _row_spec(DH2), _row_spec(DH2), _row_spec(8)],
    out_shape=[
        jax.ShapeDtypeStruct((N, H1), jnp.float32),
        jax.ShapeDtypeStruct((N, DH2), jnp.float32),
        jax.ShapeDtypeStruct((N, DH2), jnp.float32),
        jax.ShapeDtypeStruct((N, 8), jnp.float32),
    ],
)

_layer2 = pl.pallas_call(
    _l2_body,
    grid=(N // BN,),
    in_specs=[
        _row_spec(DH2), _row_spec(DH2), _row_spec(H1), _row_spec(8),
        _full_spec((H1, H2)), _full_spec((1, H2)),
        _full_spec((H2, DH3)), _full_spec((H2, DH3)),
    ],
    out_specs=[_row_spec(H2), _row_spec(DH3), _row_spec(DH3)],
    out_shape=[
        jax.ShapeDtypeStruct((N, H2), jnp.float32),
        jax.ShapeDtypeStruct((N, DH3), jnp.float32),
        jax.ShapeDtypeStruct((N, DH3), jnp.float32),
    ],
)

_layer3 = pl.pallas_call(
    _l3_body,
    grid=(N // BN,),
    in_specs=[
        _row_spec(DH3), _row_spec(DH3), _row_spec(H2), _row_spec(8),
        _full_spec((H2, C)), _full_spec((1, C)),
    ],
    out_specs=_row_spec(C),
    out_shape=jax.ShapeDtypeStruct((N, C), jnp.float32),
)


def kernel(x, edge_index, batch, Wl1, b1, Wr1, Wl2, b2, Wr2, Wl3, b3, Wr3):
  f32 = jnp.float32
  src = edge_index[0]
  dst = edge_index[1]
  pad = EPAD - E
  packed = jnp.left_shift(dst, 14) | src
  packedp = jnp.concatenate(
      [packed, jnp.full((pad,), N << 14, jnp.int32)]).reshape(NS, CHUNKS, CH)

  xa_lo = x[:, :DH1]
  xa_hi = jnp.concatenate(
      [x[:, DH1:DIN], jnp.ones((N, 1), f32),
       jnp.zeros((N, 2 * DH1 - DIN - 1), f32)], axis=1)
  p1 = _segsum_d1(xa_lo, xa_hi, packedp, jnp.zeros((ZR, DH1), f32))
  h1, u2a, u2b, inv = _layer1(x, p1[0, :N], p1[1, :N], Wl1, Wr1,
                              Wl2[:, :DH2], Wl2[:, DH2:], b1.reshape(1, H1))

  p2 = _segsum_d2(u2a, u2b, packedp, jnp.zeros((ZR, DH2), f32))
  wl3p = jnp.pad(Wl3, ((0, 0), (0, 2 * DH3 - C)))
  h2, u3a, u3b = _layer2(p2[0, :N], p2[1, :N], h1, inv, Wr2,
                         b2.reshape(1, H2), wl3p[:, :DH3], wl3p[:, DH3:])

  p3 = _segsum_d3(u3a, u3b, packedp, jnp.zeros((ZR, DH3), f32))
  return _layer3(p3[0, :N], p3[1, :N], h2, inv, Wr3, b3.reshape(1, C))


# speedup vs baseline: 7.7616x; 1.0066x over previous
import jax
import jax.numpy as jnp
from jax import lax
from jax.experimental import pallas as pl
from jax.experimental.pallas import tpu as pltpu
from jax.experimental.pallas import tpu_sc as plsc

N = 10000
E = 320000
DIN = 128
H1 = 256
H2 = 192
C = 40

NC = 2
NS = 16
CH = 128
CHUNKS = 158
EPT = CHUNKS * CH
EPAD = NS * EPT
NP = 10112
RPT = NP // NS
ZR = 64

DH1 = 72
DH2 = 96
DH3 = 32


def _make_segsum(DH, NBUF):
  HALF = NBUF // 2
  TAIL = CHUNKS % NBUF
  mesh = plsc.VectorSubcoreMesh(core_axis_name="c", subcore_axis_name="s")

  def body(ulo_hbm, uhi_hbm, packed_hbm, z_hbm, out_hbm, *scratch):
    packed_v = scratch[0]
    src_i = scratch[1:1 + NBUF]
    dst_i = scratch[1 + NBUF:1 + 2 * NBUF]
    rows = scratch[1 + 2 * NBUF:1 + 3 * NBUF]
    agg_sh = scratch[1 + 3 * NBUF]
    sg = scratch[2 + 3 * NBUF:2 + 4 * NBUF]
    ss = scratch[2 + 4 * NBUF:2 + 5 * NBUF]
    c = lax.axis_index("c")
    tid = lax.axis_index("s")
    for i in range(RPT // ZR):
      pltpu.sync_copy(z_hbm, agg_sh.at[pl.ds(tid * RPT + i * ZR, ZR)])
    rem = RPT % ZR
    if rem:
      pltpu.sync_copy(z_hbm.at[pl.ds(0, rem)],
                      agg_sh.at[pl.ds(tid * RPT + (RPT // ZR) * ZR, rem)])
    pltpu.sync_copy(packed_hbm.at[tid], packed_v)
    plsc.subcore_barrier()

    def load_idx(jj, b):
      row = packed_v.at[jj]
      for k in range(CH // 16):
        v = row[pl.ds(k * 16, 16)]
        src_i[b][pl.ds(k * 16, 16)] = v & 0x3FFF
        dst_i[b][pl.ds(k * 16, 16)] = lax.shift_right_logical(v, 14)

    def issue_gather(jj, b):
      load_idx(jj, b)

      @pl.when(c == 0)
      def _():
        pltpu.async_copy(ulo_hbm.at[src_i[b]], rows[b], sg[b])

      @pl.when(c == 1)
      def _():
        pltpu.async_copy(uhi_hbm.at[src_i[b]], rows[b], sg[b])

    for b in range(HALF):
      issue_gather(b, b)

    def outer(t, carry):
      for b in range(NBUF):
        jj = t * NBUF + b
        b2 = (b + HALF) % NBUF

        @pl.when(jj >= NBUF - HALF)
        def _(b2=b2):
          pltpu.make_async_copy(
              rows[b2], agg_sh.at[dst_i[b2]], ss[b2]).wait()

        issue_gather(jj + HALF, b2)

        pltpu.make_async_copy(ulo_hbm.at[src_i[b]], rows[b], sg[b]).wait()
        pltpu.async_copy(rows[b], agg_sh.at[dst_i[b]], ss[b], add=True)
      return carry

    lax.fori_loop(0, CHUNKS // NBUF, outer, 0)
    for jj in range(CHUNKS - TAIL, CHUNKS):
      b = jj % NBUF
      b2 = (b + HALF) % NBUF
      if jj + HALF < CHUNKS:
        pltpu.make_async_copy(rows[b2], agg_sh.at[dst_i[b2]], ss[b2]).wait()
        issue_gather(jj + HALF, b2)
      pltpu.make_async_copy(ulo_hbm.at[src_i[b]], rows[b], sg[b]).wait()
      pltpu.async_copy(rows[b], agg_sh.at[dst_i[b]], ss[b], add=True)
    for b in range(NBUF):
      pltpu.make_async_copy(rows[b], agg_sh.at[dst_i[b]], ss[b]).wait()
    plsc.subcore_barrier()
    pltpu.sync_copy(agg_sh.at[pl.ds(tid * RPT, RPT)],
                    out_hbm.at[c, pl.ds(tid * RPT, RPT)])

  return pl.kernel(
      body,
      out_type=jax.ShapeDtypeStruct((NC, NP, DH), jnp.float32),
      mesh=mesh,
      compiler_params=pltpu.CompilerParams(use_tc_tiling_on_sc=False),
      scratch_types=(
          [pltpu.VMEM((CHUNKS, CH), jnp.int32)]
          + [pltpu.VMEM((CH,), jnp.int32) for _ in range(2 * NBUF)]
          + [pltpu.VMEM((CH, DH), jnp.float32) for _ in range(NBUF)]
          + [pltpu.VMEM_SHARED((NP, DH), jnp.float32)]
          + [pltpu.SemaphoreType.DMA for _ in range(2 * NBUF)]
      ),
  )


_segsum_d1 = _make_segsum(DH1, 4)
_segsum_d2 = _make_segsum(DH2, 3)
_segsum_d3 = _make_segsum(DH3, 8)

BN = 1000


def _self_body(h_ref, w_ref, b_ref, out_ref):
  out_ref[:, :] = (
      jnp.dot(h_ref[:, :], w_ref[:, :], preferred_element_type=jnp.float32)
      + b_ref[:, :])


def _c1_body(p0_ref, p1_ref, s1_ref, wl1_ref, wl2a_ref, wl2b_ref,
             h1_ref, u2a_ref, u2b_ref, inv_ref):
  s = jnp.concatenate([p0_ref[:, :], p1_ref[:, :]], axis=1)
  cnt = s[:, DIN:DIN + 1]
  inv = 1.0 / jnp.maximum(cnt, 1.0)
  mean = s[:, :DIN] * inv
  h1 = jnp.tanh(
      jnp.dot(mean, wl1_ref[:, :], preferred_element_type=jnp.float32)
      + s1_ref[:, :])
  h1_ref[:, :] = h1
  u2a_ref[:, :] = jnp.dot(h1, wl2a_ref[:, :], preferred_element_type=jnp.float32)
  u2b_ref[:, :] = jnp.dot(h1, wl2b_ref[:, :], preferred_element_type=jnp.float32)
  inv_ref[:, :] = jnp.broadcast_to(inv, (BN, 8))


def _c2_body(p0_ref, p1_ref, s2_ref, inv_ref, wl3a_ref, wl3b_ref,
             h2_ref, u3a_ref, u3b_ref):
  agg = jnp.concatenate([p0_ref[:, :], p1_ref[:, :]], axis=1) * inv_ref[:, 0:1]
  h2 = jax.nn.relu(agg + s2_ref[:, :])
  h2_ref[:, :] = h2
  u3a_ref[:, :] = jnp.dot(h2, wl3a_ref[:, :], preferred_element_type=jnp.float32)
  u3b_ref[:, :] = jnp.dot(h2, wl3b_ref[:, :], preferred_element_type=jnp.float32)


def _c3_body(p0_ref, p1_ref, s3_ref, inv_ref, out_ref):
  agg = jnp.concatenate([p0_ref[:, :], p1_ref[:, :C - DH3]], axis=1)
  out_ref[:, :] = jax.nn.sigmoid(agg * inv_ref[:, 0:1] + s3_ref[:, :])


def _row_spec(d):
  return pl.BlockSpec((BN, d), lambda i: (i, 0))


def _full_spec(shape):
  return pl.BlockSpec(shape, lambda i: (0,) * len(shape))


def _make_self(din, dout):
  return pl.pallas_call(
      _self_body,
      grid=(N // BN,),
      in_specs=[_row_spec(din), _full_spec((din, dout)),
                _full_spec((1, dout))],
      out_specs=_row_spec(dout),
      out_shape=jax.ShapeDtypeStruct((N, dout), jnp.float32),
  )


_self1 = _make_self(DIN, H1)
_self2 = _make_self(H1, H2)
_self3 = _make_self(H2, C)

_combine1 = pl.pallas_call(
    _c1_body,
    grid=(N // BN,),
    in_specs=[
        _row_spec(DH1), _row_spec(DH1), _row_spec(H1),
        _full_spec((DIN, H1)),
        _full_spec((H1, DH2)), _full_spec((H1, DH2)),
    ],
    out_specs=[_row_spec(H1), _row_spec(DH2), _row_spec(DH2), _row_spec(8)],
    out_shape=[
        jax.ShapeDtypeStruct((N, H1), jnp.float32),
        jax.ShapeDtypeStruct((N, DH2), jnp.float32),
        jax.ShapeDtypeStruct((N, DH2), jnp.float32),
        jax.ShapeDtypeStruct((N, 8), jnp.float32),
    ],
)

_combine2 = pl.pallas_call(
    _c2_body,
    grid=(N // BN,),
    in_specs=[
        _row_spec(DH2), _row_spec(DH2), _row_spec(H2), _row_spec(8),
        _full_spec((H2, DH3)), _full_spec((H2, DH3)),
    ],
    out_specs=[_row_spec(H2), _row_spec(DH3), _row_spec(DH3)],
    out_shape=[
        jax.ShapeDtypeStruct((N, H2), jnp.float32),
        jax.ShapeDtypeStruct((N, DH3), jnp.float32),
        jax.ShapeDtypeStruct((N, DH3), jnp.float32),
    ],
)

_combine3 = pl.pallas_call(
    _c3_body,
    grid=(N // BN,),
    in_specs=[
        _row_spec(DH3), _row_spec(DH3), _row_spec(C), _row_spec(8),
    ],
    out_specs=_row_spec(C),
    out_shape=jax.ShapeDtypeStruct((N, C), jnp.float32),
)


def kernel(x, edge_index, batch, Wl1, b1, Wr1, Wl2, b2, Wr2, Wl3, b3, Wr3):
  f32 = jnp.float32
  src = edge_index[0]
  dst = edge_index[1]
  pad = EPAD - E
  packed = jnp.left_shift(dst, 14) | src
  packedp = jnp.concatenate(
      [packed, jnp.full((pad,), N << 14, jnp.int32)]).reshape(NS, CHUNKS, CH)

  xa_lo = x[:, :DH1]
  xa_hi = jnp.concatenate(
      [x[:, DH1:DIN], jnp.ones((N, 1), f32),
       jnp.zeros((N, 2 * DH1 - DIN - 1), f32)], axis=1)
  p1 = _segsum_d1(xa_lo, xa_hi, packedp, jnp.zeros((ZR, DH1), f32))
  s1 = _self1(x, Wr1, b1.reshape(1, H1))
  h1, u2a, u2b, inv = _combine1(p1[0, :N], p1[1, :N], s1, Wl1,
                                Wl2[:, :DH2], Wl2[:, DH2:])

  p2 = _segsum_d2(u2a, u2b, packedp, jnp.zeros((ZR, DH2), f32))
  s2 = _self2(h1, Wr2, b2.reshape(1, H2))
  wl3p = jnp.pad(Wl3, ((0, 0), (0, 2 * DH3 - C)))
  h2, u3a, u3b = _combine2(p2[0, :N], p2[1, :N], s2, inv,
                           wl3p[:, :DH3], wl3p[:, DH3:])

  p3 = _segsum_d3(u3a, u3b, packedp, jnp.zeros((ZR, DH3), f32))
  s3 = _self3(h2, Wr3, b3.reshape(1, C))
  return _combine3(p3[0, :N], p3[1, :N], s3, inv)


# speedup vs baseline: 7.8906x; 1.0166x over previous
import jax
import jax.numpy as jnp
from jax import lax
from jax.experimental import pallas as pl
from jax.experimental.pallas import tpu as pltpu
from jax.experimental.pallas import tpu_sc as plsc

N = 10000
E = 320000
DIN = 128
H1 = 256
H2 = 192
C = 40

NC = 2
NS = 16
CH = 128
CHUNKS = 158
EPT = CHUNKS * CH
EPAD = NS * EPT
NP = 10112
RPT = NP // NS
ZR = 64

DH1 = 72
DH2 = 96
DH3 = 24


def _make_segsum(DH, NBUF):
  HALF = NBUF // 2
  TAIL = CHUNKS % NBUF
  mesh = plsc.VectorSubcoreMesh(core_axis_name="c", subcore_axis_name="s")

  def body(ulo_hbm, uhi_hbm, packed_hbm, z_hbm, out_hbm, *scratch):
    packed_v = scratch[0]
    src_i = scratch[1:1 + NBUF]
    dst_i = scratch[1 + NBUF:1 + 2 * NBUF]
    rows = scratch[1 + 2 * NBUF:1 + 3 * NBUF]
    agg_sh = scratch[1 + 3 * NBUF]
    sg = scratch[2 + 3 * NBUF:2 + 4 * NBUF]
    ss = scratch[2 + 4 * NBUF:2 + 5 * NBUF]
    c = lax.axis_index("c")
    tid = lax.axis_index("s")
    for i in range(RPT // ZR):
      pltpu.sync_copy(z_hbm, agg_sh.at[pl.ds(tid * RPT + i * ZR, ZR)])
    rem = RPT % ZR
    if rem:
      pltpu.sync_copy(z_hbm.at[pl.ds(0, rem)],
                      agg_sh.at[pl.ds(tid * RPT + (RPT // ZR) * ZR, rem)])
    pltpu.sync_copy(packed_hbm.at[tid], packed_v)
    plsc.subcore_barrier()

    def load_idx(jj, b):
      row = packed_v.at[jj]
      for k in range(CH // 16):
        v = row[pl.ds(k * 16, 16)]
        src_i[b][pl.ds(k * 16, 16)] = v & 0x3FFF
        dst_i[b][pl.ds(k * 16, 16)] = lax.shift_right_logical(v, 14)

    def issue_gather(jj, b):
      load_idx(jj, b)

      @pl.when(c == 0)
      def _():
        pltpu.async_copy(ulo_hbm.at[src_i[b]], rows[b], sg[b])

      @pl.when(c == 1)
      def _():
        pltpu.async_copy(uhi_hbm.at[src_i[b]], rows[b], sg[b])

    for b in range(HALF):
      issue_gather(b, b)

    def outer(t, carry):
      for b in range(NBUF):
        jj = t * NBUF + b
        b2 = (b + HALF) % NBUF

        @pl.when(jj >= NBUF - HALF)
        def _(b2=b2):
          pltpu.make_async_copy(
              rows[b2], agg_sh.at[dst_i[b2]], ss[b2]).wait()

        issue_gather(jj + HALF, b2)

        pltpu.make_async_copy(ulo_hbm.at[src_i[b]], rows[b], sg[b]).wait()
        pltpu.async_copy(rows[b], agg_sh.at[dst_i[b]], ss[b], add=True)
      return carry

    lax.fori_loop(0, CHUNKS // NBUF, outer, 0)
    for jj in range(CHUNKS - TAIL, CHUNKS):
      b = jj % NBUF
      b2 = (b + HALF) % NBUF
      if jj + HALF < CHUNKS:
        pltpu.make_async_copy(rows[b2], agg_sh.at[dst_i[b2]], ss[b2]).wait()
        issue_gather(jj + HALF, b2)
      pltpu.make_async_copy(ulo_hbm.at[src_i[b]], rows[b], sg[b]).wait()
      pltpu.async_copy(rows[b], agg_sh.at[dst_i[b]], ss[b], add=True)
    for b in range(NBUF):
      pltpu.make_async_copy(rows[b], agg_sh.at[dst_i[b]], ss[b]).wait()
    plsc.subcore_barrier()
    pltpu.sync_copy(agg_sh.at[pl.ds(tid * RPT, RPT)],
                    out_hbm.at[c, pl.ds(tid * RPT, RPT)])

  return pl.kernel(
      body,
      out_type=jax.ShapeDtypeStruct((NC, NP, DH), jnp.float32),
      mesh=mesh,
      compiler_params=pltpu.CompilerParams(use_tc_tiling_on_sc=False),
      scratch_types=(
          [pltpu.VMEM((CHUNKS, CH), jnp.int32)]
          + [pltpu.VMEM((CH,), jnp.int32) for _ in range(2 * NBUF)]
          + [pltpu.VMEM((CH, DH), jnp.float32) for _ in range(NBUF)]
          + [pltpu.VMEM_SHARED((NP, DH), jnp.float32)]
          + [pltpu.SemaphoreType.DMA for _ in range(2 * NBUF)]
      ),
  )


_segsum_d1 = _make_segsum(DH1, 4)
_segsum_d2 = _make_segsum(DH2, 3)
_segsum_d3 = _make_segsum(DH3, 8)

BN = 1000


def _self_body(h_ref, w_ref, b_ref, out_ref):
  out_ref[:, :] = (
      jnp.dot(h_ref[:, :], w_ref[:, :], preferred_element_type=jnp.float32)
      + b_ref[:, :])


def _c1_body(p0_ref, p1_ref, s1_ref, wl1_ref, wl2a_ref, wl2b_ref,
             h1_ref, u2a_ref, u2b_ref, inv_ref):
  s = jnp.concatenate([p0_ref[:, :], p1_ref[:, :]], axis=1)
  cnt = s[:, DH1 - 1:DH1]
  inv = 1.0 / jnp.maximum(cnt, 1.0)
  mean = jnp.concatenate(
      [s[:, :DH1 - 1], s[:, DH1:DH1 + DIN - DH1 + 1]], axis=1) * inv
  h1 = jnp.tanh(
      jnp.dot(mean, wl1_ref[:, :], preferred_element_type=jnp.float32)
      + s1_ref[:, :])
  h1_ref[:, :] = h1
  u2a_ref[:, :] = jnp.dot(h1, wl2a_ref[:, :], preferred_element_type=jnp.float32)
  u2b_ref[:, :] = jnp.dot(h1, wl2b_ref[:, :], preferred_element_type=jnp.float32)
  inv_ref[:, :] = jnp.broadcast_to(inv, (BN, 8))


def _c2_body(p0_ref, p1_ref, s2_ref, inv_ref, wl3a_ref, wl3b_ref,
             h2_ref, u3a_ref, u3b_ref):
  agg = jnp.concatenate([p0_ref[:, :], p1_ref[:, :]], axis=1) * inv_ref[:, 0:1]
  h2 = jax.nn.relu(agg + s2_ref[:, :])
  h2_ref[:, :] = h2
  u3a_ref[:, :] = jnp.dot(h2, wl3a_ref[:, :], preferred_element_type=jnp.float32)
  u3b_ref[:, :] = jnp.dot(h2, wl3b_ref[:, :], preferred_element_type=jnp.float32)


def _c3_body(p0_ref, p1_ref, s3_ref, inv_ref, out_ref):
  agg = jnp.concatenate([p0_ref[:, :], p1_ref[:, :C - DH3]], axis=1)
  out_ref[:, :] = jax.nn.sigmoid(agg * inv_ref[:, 0:1] + s3_ref[:, :])


def _row_spec(d):
  return pl.BlockSpec((BN, d), lambda i: (i, 0))


def _full_spec(shape):
  return pl.BlockSpec(shape, lambda i: (0,) * len(shape))


def _make_self(din, dout):
  return pl.pallas_call(
      _self_body,
      grid=(N // BN,),
      in_specs=[_row_spec(din), _full_spec((din, dout)),
                _full_spec((1, dout))],
      out_specs=_row_spec(dout),
      out_shape=jax.ShapeDtypeStruct((N, dout), jnp.float32),
  )


_self1 = _make_self(DIN, H1)
_self2 = _make_self(H1, H2)
_self3 = _make_self(H2, C)

_combine1 = pl.pallas_call(
    _c1_body,
    grid=(N // BN,),
    in_specs=[
        _row_spec(DH1), _row_spec(DH1), _row_spec(H1),
        _full_spec((DIN, H1)),
        _full_spec((H1, DH2)), _full_spec((H1, DH2)),
    ],
    out_specs=[_row_spec(H1), _row_spec(DH2), _row_spec(DH2), _row_spec(8)],
    out_shape=[
        jax.ShapeDtypeStruct((N, H1), jnp.float32),
        jax.ShapeDtypeStruct((N, DH2), jnp.float32),
        jax.ShapeDtypeStruct((N, DH2), jnp.float32),
        jax.ShapeDtypeStruct((N, 8), jnp.float32),
    ],
)

_combine2 = pl.pallas_call(
    _c2_body,
    grid=(N // BN,),
    in_specs=[
        _row_spec(DH2), _row_spec(DH2), _row_spec(H2), _row_spec(8),
        _full_spec((H2, DH3)), _full_spec((H2, DH3)),
    ],
    out_specs=[_row_spec(H2), _row_spec(DH3), _row_spec(DH3)],
    out_shape=[
        jax.ShapeDtypeStruct((N, H2), jnp.float32),
        jax.ShapeDtypeStruct((N, DH3), jnp.float32),
        jax.ShapeDtypeStruct((N, DH3), jnp.float32),
    ],
)

_combine3 = pl.pallas_call(
    _c3_body,
    grid=(N // BN,),
    in_specs=[
        _row_spec(DH3), _row_spec(DH3), _row_spec(C), _row_spec(8),
    ],
    out_specs=_row_spec(C),
    out_shape=jax.ShapeDtypeStruct((N, C), jnp.float32),
)


def kernel(x, edge_index, batch, Wl1, b1, Wr1, Wl2, b2, Wr2, Wl3, b3, Wr3):
  f32 = jnp.float32
  src = edge_index[0]
  dst = edge_index[1]
  pad = EPAD - E
  packed = jnp.left_shift(dst, 14) | src
  packedp = jnp.concatenate(
      [packed, jnp.full((pad,), N << 14, jnp.int32)]).reshape(NS, CHUNKS, CH)

  xa_lo = jnp.concatenate([x[:, :DH1 - 1], jnp.ones((N, 1), f32)], axis=1)
  xa_hi = jnp.concatenate(
      [x[:, DH1 - 1:DIN],
       jnp.zeros((N, 2 * DH1 - DIN - 1), f32)], axis=1)
  p1 = _segsum_d1(xa_lo, xa_hi, packedp, jnp.zeros((ZR, DH1), f32))
  s1 = _self1(x, Wr1, b1.reshape(1, H1))
  h1, u2a, u2b, inv = _combine1(p1[0, :N], p1[1, :N], s1, Wl1,
                                Wl2[:, :DH2], Wl2[:, DH2:])

  p2 = _segsum_d2(u2a, u2b, packedp, jnp.zeros((ZR, DH2), f32))
  s2 = _self2(h1, Wr2, b2.reshape(1, H2))
  wl3p = jnp.pad(Wl3, ((0, 0), (0, 2 * DH3 - C)))
  h2, u3a, u3b = _combine2(p2[0, :N], p2[1, :N], s2, inv,
                           wl3p[:, :DH3], wl3p[:, DH3:])

  p3 = _segsum_d3(u3a, u3b, packedp, jnp.zeros((ZR, DH3), f32))
  s3 = _self3(h2, Wr3, b3.reshape(1, C))
  return _combine3(p3[0, :N], p3[1, :N], s3, inv)


# speedup vs baseline: 8.0548x; 1.0208x over previous
import jax
import jax.numpy as jnp
from jax import lax
from jax.experimental import pallas as pl
from jax.experimental.pallas import tpu as pltpu
from jax.experimental.pallas import tpu_sc as plsc

N = 10000
E = 320000
DIN = 128
H1 = 256
H2 = 192
C = 40

NC = 2
NS = 16
CH = 128
CHUNKS = 158
EPT = CHUNKS * CH
EPAD = NS * EPT
NP = 10112
RPT = NP // NS
ZR = 64

DH1 = 72
DH2 = 96
DH3 = 24


def _make_segsum(DH, NBUF):
  HALF = NBUF // 2
  TAIL = CHUNKS % NBUF
  LASTR = N - (NS - 1) * RPT
  mesh = plsc.VectorSubcoreMesh(core_axis_name="c", subcore_axis_name="s")

  def body(u_hbm, packed_hbm, z_hbm, out_hbm, *scratch):
    packed_v = scratch[0]
    src_i = scratch[1:1 + NBUF]
    dst_i = scratch[1 + NBUF:1 + 2 * NBUF]
    rows = scratch[1 + 2 * NBUF:1 + 3 * NBUF]
    agg_sh = scratch[1 + 3 * NBUF]
    sg = scratch[2 + 3 * NBUF:2 + 4 * NBUF]
    ss = scratch[2 + 4 * NBUF:2 + 5 * NBUF]
    c = lax.axis_index("c")
    tid = lax.axis_index("s")
    for i in range(RPT // ZR):
      pltpu.sync_copy(z_hbm, agg_sh.at[pl.ds(tid * RPT + i * ZR, ZR)])
    rem = RPT % ZR
    if rem:
      pltpu.sync_copy(z_hbm.at[pl.ds(0, rem)],
                      agg_sh.at[pl.ds(tid * RPT + (RPT // ZR) * ZR, rem)])
    pltpu.sync_copy(packed_hbm.at[tid], packed_v)
    plsc.subcore_barrier()

    def load_idx(jj, b):
      row = packed_v.at[jj]
      for k in range(CH // 16):
        v = row[pl.ds(k * 16, 16)]
        src_i[b][pl.ds(k * 16, 16)] = v & 0x3FFF
        dst_i[b][pl.ds(k * 16, 16)] = lax.shift_right_logical(v, 14)

    def issue_gather(jj, b):
      load_idx(jj, b)
      pltpu.async_copy(u_hbm.at[c].at[src_i[b]], rows[b], sg[b])

    for b in range(HALF):
      issue_gather(b, b)

    def outer(t, carry):
      for b in range(NBUF):
        jj = t * NBUF + b
        b2 = (b + HALF) % NBUF

        @pl.when(jj >= NBUF - HALF)
        def _(b2=b2):
          pltpu.make_async_copy(
              rows[b2], agg_sh.at[dst_i[b2]], ss[b2]).wait()

        issue_gather(jj + HALF, b2)

        pltpu.make_async_copy(u_hbm.at[c].at[src_i[b]], rows[b], sg[b]).wait()
        pltpu.async_copy(rows[b], agg_sh.at[dst_i[b]], ss[b], add=True)
      return carry

    lax.fori_loop(0, CHUNKS // NBUF, outer, 0)
    for jj in range(CHUNKS - TAIL, CHUNKS):
      b = jj % NBUF
      b2 = (b + HALF) % NBUF
      if jj + HALF < CHUNKS:
        pltpu.make_async_copy(rows[b2], agg_sh.at[dst_i[b2]], ss[b2]).wait()
        issue_gather(jj + HALF, b2)
      pltpu.make_async_copy(u_hbm.at[c].at[src_i[b]], rows[b], sg[b]).wait()
      pltpu.async_copy(rows[b], agg_sh.at[dst_i[b]], ss[b], add=True)
    for b in range(NBUF):
      pltpu.make_async_copy(rows[b], agg_sh.at[dst_i[b]], ss[b]).wait()
    plsc.subcore_barrier()
    @pl.when(tid < NS - 1)
    def _():
      pltpu.sync_copy(agg_sh.at[pl.ds(tid * RPT, RPT)],
                      out_hbm.at[c, pl.ds(tid * RPT, RPT)])

    @pl.when(tid == NS - 1)
    def _():
      pltpu.sync_copy(agg_sh.at[pl.ds((NS - 1) * RPT, LASTR)],
                      out_hbm.at[c, pl.ds((NS - 1) * RPT, LASTR)])

  return pl.kernel(
      body,
      out_type=jax.ShapeDtypeStruct((NC, N, DH), jnp.float32),
      mesh=mesh,
      compiler_params=pltpu.CompilerParams(use_tc_tiling_on_sc=False),
      scratch_types=(
          [pltpu.VMEM((CHUNKS, CH), jnp.int32)]
          + [pltpu.VMEM((CH,), jnp.int32) for _ in range(2 * NBUF)]
          + [pltpu.VMEM((CH, DH), jnp.float32) for _ in range(NBUF)]
          + [pltpu.VMEM_SHARED((NP, DH), jnp.float32)]
          + [pltpu.SemaphoreType.DMA for _ in range(2 * NBUF)]
      ),
  )


_segsum_d1 = _make_segsum(DH1, 4)
_segsum_d2 = _make_segsum(DH2, 3)
_segsum_d3 = _make_segsum(DH3, 8)

BN = 1000


def _self_body(h_ref, w_ref, b_ref, out_ref):
  out_ref[:, :] = (
      jnp.dot(h_ref[:, :], w_ref[:, :], preferred_element_type=jnp.float32)
      + b_ref[:, :])


def _c1_body(p0_ref, p1_ref, s1_ref, wl1_ref, wl2a_ref, wl2b_ref,
             h1_ref, u2_ref, inv_ref):
  s = jnp.concatenate([p0_ref[0], p1_ref[0]], axis=1)
  cnt = s[:, DH1 - 1:DH1]
  inv = 1.0 / jnp.maximum(cnt, 1.0)
  mean = jnp.concatenate(
      [s[:, :DH1 - 1], s[:, DH1:DH1 + DIN - DH1 + 1]], axis=1) * inv
  h1 = jnp.tanh(
      jnp.dot(mean, wl1_ref[:, :], preferred_element_type=jnp.float32)
      + s1_ref[:, :])
  h1_ref[:, :] = h1
  u2_ref[0, :, :] = jnp.dot(h1, wl2a_ref[:, :],
                            preferred_element_type=jnp.float32)
  u2_ref[1, :, :] = jnp.dot(h1, wl2b_ref[:, :],
                            preferred_element_type=jnp.float32)
  inv_ref[:, :] = jnp.broadcast_to(inv, (BN, 8))


def _c2_body(p0_ref, p1_ref, s2_ref, inv_ref, wl3a_ref, wl3b_ref,
             h2_ref, u3_ref):
  agg = jnp.concatenate([p0_ref[0], p1_ref[0]], axis=1) * inv_ref[:, 0:1]
  h2 = jax.nn.relu(agg + s2_ref[:, :])
  h2_ref[:, :] = h2
  u3_ref[0, :, :] = jnp.dot(h2, wl3a_ref[:, :],
                            preferred_element_type=jnp.float32)
  u3_ref[1, :, :] = jnp.dot(h2, wl3b_ref[:, :],
                            preferred_element_type=jnp.float32)


def _c3_body(p0_ref, p1_ref, s3_ref, inv_ref, out_ref):
  agg = jnp.concatenate([p0_ref[0], p1_ref[0][:, :C - DH3]], axis=1)
  out_ref[:, :] = jax.nn.sigmoid(agg * inv_ref[:, 0:1] + s3_ref[:, :])


def _row_spec(d):
  return pl.BlockSpec((BN, d), lambda i: (i, 0))


def _core_spec(d, k):
  return pl.BlockSpec((1, BN, d), lambda i, k=k: (k, i, 0))


def _stack_spec(d):
  return pl.BlockSpec((NC, BN, d), lambda i: (0, i, 0))


def _full_spec(shape):
  return pl.BlockSpec(shape, lambda i: (0,) * len(shape))


def _make_self(din, dout):
  return pl.pallas_call(
      _self_body,
      grid=(N // BN,),
      in_specs=[_row_spec(din), _full_spec((din, dout)),
                _full_spec((1, dout))],
      out_specs=_row_spec(dout),
      out_shape=jax.ShapeDtypeStruct((N, dout), jnp.float32),
  )


_self1 = _make_self(DIN, H1)
_self2 = _make_self(H1, H2)
_self3 = _make_self(H2, C)

_combine1 = pl.pallas_call(
    _c1_body,
    grid=(N // BN,),
    in_specs=[
        _core_spec(DH1, 0), _core_spec(DH1, 1), _row_spec(H1),
        _full_spec((DIN, H1)),
        _full_spec((H1, DH2)), _full_spec((H1, DH2)),
    ],
    out_specs=[_row_spec(H1), _stack_spec(DH2), _row_spec(8)],
    out_shape=[
        jax.ShapeDtypeStruct((N, H1), jnp.float32),
        jax.ShapeDtypeStruct((NC, N, DH2), jnp.float32),
        jax.ShapeDtypeStruct((N, 8), jnp.float32),
    ],
)

_combine2 = pl.pallas_call(
    _c2_body,
    grid=(N // BN,),
    in_specs=[
        _core_spec(DH2, 0), _core_spec(DH2, 1), _row_spec(H2), _row_spec(8),
        _full_spec((H2, DH3)), _full_spec((H2, DH3)),
    ],
    out_specs=[_row_spec(H2), _stack_spec(DH3)],
    out_shape=[
        jax.ShapeDtypeStruct((N, H2), jnp.float32),
        jax.ShapeDtypeStruct((NC, N, DH3), jnp.float32),
    ],
)

_combine3 = pl.pallas_call(
    _c3_body,
    grid=(N // BN,),
    in_specs=[
        _core_spec(DH3, 0), _core_spec(DH3, 1), _row_spec(C), _row_spec(8),
    ],
    out_specs=_row_spec(C),
    out_shape=jax.ShapeDtypeStruct((N, C), jnp.float32),
)


def kernel(x, edge_index, batch, Wl1, b1, Wr1, Wl2, b2, Wr2, Wl3, b3, Wr3):
  f32 = jnp.float32
  src = edge_index[0]
  dst = edge_index[1]
  pad = EPAD - E
  packed = jnp.left_shift(dst, 14) | src
  packedp = jnp.concatenate(
      [packed, jnp.full((pad,), N << 14, jnp.int32)]).reshape(NS, CHUNKS, CH)

  xa = jnp.stack([
      jnp.concatenate([x[:, :DH1 - 1], jnp.ones((N, 1), f32)], axis=1),
      jnp.concatenate([x[:, DH1 - 1:DIN],
                       jnp.zeros((N, 2 * DH1 - DIN - 1), f32)], axis=1),
  ])
  p1 = _segsum_d1(xa, packedp, jnp.zeros((ZR, DH1), f32))
  s1 = _self1(x, Wr1, b1.reshape(1, H1))
  h1, u2, inv = _combine1(p1, p1, s1, Wl1, Wl2[:, :DH2], Wl2[:, DH2:])

  p2 = _segsum_d2(u2, packedp, jnp.zeros((ZR, DH2), f32))
  s2 = _self2(h1, Wr2, b2.reshape(1, H2))
  wl3p = jnp.pad(Wl3, ((0, 0), (0, 2 * DH3 - C)))
  h2, u3 = _combine2(p2, p2, s2, inv, wl3p[:, :DH3], wl3p[:, DH3:])

  p3 = _segsum_d3(u3, packedp, jnp.zeros((ZR, DH3), f32))
  s3 = _self3(h2, Wr3, b3.reshape(1, C))
  return _combine3(p3, p3, s3, inv)
